# initial kernel scaffold (unmeasured)
import jax
import jax.numpy as jnp
from jax import lax
from jax.experimental import pallas as pl
from jax.experimental.pallas import tpu as pltpu

N_DEV = 8
M = 4096
N_OUT = 2048
CHUNK = M // N_DEV


def _silu(v):
    return v * (1.0 / (1.0 + jnp.exp(-v)))


def _ar_body(p_ref, out_ref, comm_ref, send_sems, recv_sems, credit_sem):
    my = lax.axis_index("i")
    left = lax.rem(my + N_DEV - 1, N_DEV)
    right = lax.rem(my + 1, N_DEV)

    barrier_sem = pltpu.get_barrier_semaphore()
    for nbr in (left, right):
        pl.semaphore_signal(
            barrier_sem, inc=1,
            device_id=(nbr,), device_id_type=pl.DeviceIdType.MESH,
        )
    pl.semaphore_wait(barrier_sem, 2)

    comm_ref[0] = p_ref[pl.ds(my * CHUNK, CHUNK), :]

    n_steps = 2 * (N_DEV - 1)
    for t in range(n_steps):
        s_slot = t % 2
        r_slot = (t + 1) % 2
        if t >= 1:
            pl.semaphore_wait(credit_sem, 1)
        rdma = pltpu.make_async_remote_copy(
            src_ref=comm_ref.at[s_slot],
            dst_ref=comm_ref.at[r_slot],
            send_sem=send_sems.at[s_slot],
            recv_sem=recv_sems.at[r_slot],
            device_id=(right,),
            device_id_type=pl.DeviceIdType.MESH,
        )
        rdma.start()
        rdma.wait()
        if t < n_steps - 1:
            pl.semaphore_signal(
                credit_sem, inc=1,
                device_id=(left,), device_id_type=pl.DeviceIdType.MESH,
            )
        if t < N_DEV - 2:
            c = lax.rem(my - (t + 1) + N_DEV, N_DEV)
            comm_ref[r_slot] = comm_ref[r_slot] + p_ref[pl.ds(c * CHUNK, CHUNK), :]
        elif t == N_DEV - 2:
            c = lax.rem(my + 1, N_DEV)
            v = _silu(comm_ref[r_slot] + p_ref[pl.ds(c * CHUNK, CHUNK), :])
            comm_ref[r_slot] = v
            out_ref[pl.ds(c * CHUNK, CHUNK), :] = v
        else:
            c = lax.rem(my - (t - (N_DEV - 1)) + N_DEV, N_DEV)
            out_ref[pl.ds(c * CHUNK, CHUNK), :] = comm_ref[r_slot]


def _ring_all_reduce_silu(partial):
    return pl.pallas_call(
        _ar_body,
        out_shape=jax.ShapeDtypeStruct((M, N_OUT), jnp.float32),
        in_specs=[pl.BlockSpec(memory_space=pltpu.VMEM)],
        out_specs=pl.BlockSpec(memory_space=pltpu.VMEM),
        scratch_shapes=[
            pltpu.VMEM((2, CHUNK, N_OUT), jnp.float32),
            pltpu.SemaphoreType.DMA((2,)),
            pltpu.SemaphoreType.DMA((2,)),
            pltpu.SemaphoreType.REGULAR,
        ],
        compiler_params=pltpu.CompilerParams(collective_id=0),
    )(partial)


def kernel(x, w_mat):
    partial = jnp.dot(x, w_mat, preferred_element_type=jnp.float32)
    return _ring_all_reduce_silu(partial)


# baseline (device time: 742440 ns/iter reference)
import jax
import jax.numpy as jnp
from jax import lax
from jax.experimental import pallas as pl
from jax.experimental.pallas import tpu as pltpu

N_DEV = 8
M = 4096
N_OUT = 2048
CHUNK = M // N_DEV


def _silu(v):
    return v * (1.0 / (1.0 + jnp.exp(-v)))


def _ar_body(p_hbm, out_hbm, comm_ref, pstage, send_sems, recv_sems,
             local_sem, out_sem, credit_sem):
    my = lax.axis_index("i")
    left = lax.rem(my + N_DEV - 1, N_DEV)
    right = lax.rem(my + 1, N_DEV)

    barrier_sem = pltpu.get_barrier_semaphore()
    for nbr in (left, right):
        pl.semaphore_signal(
            barrier_sem, inc=1,
            device_id=(nbr,), device_id_type=pl.DeviceIdType.MESH,
        )
    pl.semaphore_wait(barrier_sem, 2)

    init = pltpu.make_async_copy(
        p_hbm.at[pl.ds(my * CHUNK, CHUNK), :], comm_ref.at[0], local_sem)
    init.start()
    init.wait()

    n_steps = 2 * (N_DEV - 1)
    for t in range(n_steps):
        s_slot = t % 2
        r_slot = (t + 1) % 2
        if t >= 1:
            pl.semaphore_wait(credit_sem, 1)
        rdma = pltpu.make_async_remote_copy(
            src_ref=comm_ref.at[s_slot],
            dst_ref=comm_ref.at[r_slot],
            send_sem=send_sems.at[s_slot],
            recv_sem=recv_sems.at[r_slot],
            device_id=(right,),
            device_id_type=pl.DeviceIdType.MESH,
        )
        rdma.start()
        if t < N_DEV - 1:
            c = lax.rem(my - (t + 1) + N_DEV, N_DEV)
            fetch = pltpu.make_async_copy(
                p_hbm.at[pl.ds(c * CHUNK, CHUNK), :], pstage, local_sem)
            fetch.start()
            fetch.wait()
        rdma.wait()
        if t < N_DEV - 2:
            comm_ref[r_slot] = comm_ref[r_slot] + pstage[...]
        elif t == N_DEV - 2:
            comm_ref[r_slot] = _silu(comm_ref[r_slot] + pstage[...])
        if t >= N_DEV - 2:
            c = lax.rem(my - (t - (N_DEV - 1)) + N_DEV, N_DEV)
            store = pltpu.make_async_copy(
                comm_ref.at[r_slot], out_hbm.at[pl.ds(c * CHUNK, CHUNK), :],
                out_sem)
            store.start()
            store.wait()
        if t < n_steps - 1:
            pl.semaphore_signal(
                credit_sem, inc=1,
                device_id=(left,), device_id_type=pl.DeviceIdType.MESH,
            )


def _ring_all_reduce_silu(partial):
    return pl.pallas_call(
        _ar_body,
        out_shape=jax.ShapeDtypeStruct((M, N_OUT), jnp.float32),
        in_specs=[pl.BlockSpec(memory_space=pltpu.MemorySpace.HBM)],
        out_specs=pl.BlockSpec(memory_space=pltpu.MemorySpace.HBM),
        scratch_shapes=[
            pltpu.VMEM((2, CHUNK, N_OUT), jnp.float32),
            pltpu.VMEM((CHUNK, N_OUT), jnp.float32),
            pltpu.SemaphoreType.DMA((2,)),
            pltpu.SemaphoreType.DMA((2,)),
            pltpu.SemaphoreType.DMA,
            pltpu.SemaphoreType.DMA,
            pltpu.SemaphoreType.REGULAR,
        ],
        compiler_params=pltpu.CompilerParams(collective_id=0),
    )(partial)


def kernel(x, w_mat):
    partial = jnp.dot(x, w_mat, preferred_element_type=jnp.float32)
    return _ring_all_reduce_silu(partial)


# device time: 413702 ns/iter; 1.7946x vs baseline; 1.7946x over previous
import jax
import jax.numpy as jnp
from jax import lax
from jax.experimental import pallas as pl
from jax.experimental.pallas import tpu as pltpu

N_DEV = 8
M = 4096
K_SHARD = 512
N_OUT = 2048
HALF = N_OUT // 2
CHUNK = M // N_DEV

N_STEPS = 2 * (N_DEV - 1)


def _silu(v):
    return v * (1.0 / (1.0 + jnp.exp(-v)))


def _body(x_ref, w_ref, out_hbm, comm_cw, comm_ccw,
          send_cw, recv_cw, send_ccw, recv_ccw,
          out_sem_a, out_sem_b, credit_cw, credit_ccw):
    my = lax.axis_index("i")
    left = lax.rem(my + N_DEV - 1, N_DEV)
    right = lax.rem(my + 1, N_DEV)

    def pchunk(c, lo, hi):
        return jnp.dot(
            x_ref[pl.ds(c * CHUNK, CHUNK), :], w_ref[:, lo:hi],
            preferred_element_type=jnp.float32,
        )

    barrier_sem = pltpu.get_barrier_semaphore()
    for nbr in (left, right):
        pl.semaphore_signal(
            barrier_sem, inc=1,
            device_id=(nbr,), device_id_type=pl.DeviceIdType.MESH,
        )
    pl.semaphore_wait(barrier_sem, 2)

    comm_cw[0] = pchunk(my, 0, HALF)
    comm_ccw[0] = pchunk(my, HALF, N_OUT)

    for t in range(N_STEPS):
        s_slot = t % 2
        r_slot = (t + 1) % 2
        if t >= 1:
            pl.semaphore_wait(credit_cw, 1)
            pl.semaphore_wait(credit_ccw, 1)
        rdma_cw = pltpu.make_async_remote_copy(
            src_ref=comm_cw.at[s_slot], dst_ref=comm_cw.at[r_slot],
            send_sem=send_cw.at[s_slot], recv_sem=recv_cw.at[r_slot],
            device_id=(right,), device_id_type=pl.DeviceIdType.MESH,
        )
        rdma_ccw = pltpu.make_async_remote_copy(
            src_ref=comm_ccw.at[s_slot], dst_ref=comm_ccw.at[r_slot],
            send_sem=send_ccw.at[s_slot], recv_sem=recv_ccw.at[r_slot],
            device_id=(left,), device_id_type=pl.DeviceIdType.MESH,
        )
        rdma_cw.start()
        rdma_ccw.start()
        if t < N_DEV - 1:
            c_cw = lax.rem(my - (t + 1) + N_DEV, N_DEV)
            c_ccw = lax.rem(my + t + 1, N_DEV)
            pa = pchunk(c_cw, 0, HALF)
            pb = pchunk(c_ccw, HALF, N_OUT)
        rdma_cw.wait()
        rdma_ccw.wait()
        if t < N_DEV - 2:
            comm_cw[r_slot] = comm_cw[r_slot] + pa
            comm_ccw[r_slot] = comm_ccw[r_slot] + pb
        elif t == N_DEV - 2:
            comm_cw[r_slot] = _silu(comm_cw[r_slot] + pa)
            comm_ccw[r_slot] = _silu(comm_ccw[r_slot] + pb)
        if t >= N_DEV - 2:
            c_a = lax.rem(my - (t - (N_DEV - 1)) + N_DEV, N_DEV)
            c_b = lax.rem(my + (t - (N_DEV - 1)) + N_DEV, N_DEV)
            st_a = pltpu.make_async_copy(
                comm_cw.at[r_slot],
                out_hbm.at[pl.ds(c_a * CHUNK, CHUNK), pl.ds(0, HALF)],
                out_sem_a)
            st_b = pltpu.make_async_copy(
                comm_ccw.at[r_slot],
                out_hbm.at[pl.ds(c_b * CHUNK, CHUNK), pl.ds(HALF, HALF)],
                out_sem_b)
            st_a.start()
            st_b.start()
            st_a.wait()
            st_b.wait()
        if t < N_STEPS - 1:
            pl.semaphore_signal(
                credit_cw, inc=1,
                device_id=(left,), device_id_type=pl.DeviceIdType.MESH,
            )
            pl.semaphore_signal(
                credit_ccw, inc=1,
                device_id=(right,), device_id_type=pl.DeviceIdType.MESH,
            )


def kernel(x, w_mat):
    return pl.pallas_call(
        _body,
        out_shape=jax.ShapeDtypeStruct((M, N_OUT), jnp.float32),
        in_specs=[
            pl.BlockSpec(memory_space=pltpu.VMEM),
            pl.BlockSpec(memory_space=pltpu.VMEM),
        ],
        out_specs=pl.BlockSpec(memory_space=pltpu.MemorySpace.HBM),
        scratch_shapes=[
            pltpu.VMEM((2, CHUNK, HALF), jnp.float32),
            pltpu.VMEM((2, CHUNK, HALF), jnp.float32),
            pltpu.SemaphoreType.DMA((2,)),
            pltpu.SemaphoreType.DMA((2,)),
            pltpu.SemaphoreType.DMA((2,)),
            pltpu.SemaphoreType.DMA((2,)),
            pltpu.SemaphoreType.DMA,
            pltpu.SemaphoreType.DMA,
            pltpu.SemaphoreType.REGULAR,
            pltpu.SemaphoreType.REGULAR,
        ],
        compiler_params=pltpu.CompilerParams(collective_id=0),
    )(x, w_mat)


# device time: 396590 ns/iter; 1.8721x vs baseline; 1.0431x over previous
import jax
import jax.numpy as jnp
from jax import lax
from jax.experimental import pallas as pl
from jax.experimental.pallas import tpu as pltpu

N_DEV = 8
M = 4096
K_SHARD = 512
N_OUT = 2048
HALF = N_OUT // 2
CHUNK = M // N_DEV

N_STEPS = 2 * (N_DEV - 1)


def _silu(v):
    return v * (1.0 / (1.0 + jnp.exp(-v)))


def _body(x_ref, w_ref, out_hbm, comm_cw, comm_ccw,
          send_cw, recv_cw, send_ccw, recv_ccw,
          out_sem_a, out_sem_b, credit_cw, credit_ccw):
    my = lax.axis_index("i")
    left = lax.rem(my + N_DEV - 1, N_DEV)
    right = lax.rem(my + 1, N_DEV)

    def pchunk(c, lo, hi):
        return jnp.dot(
            x_ref[pl.ds(c * CHUNK, CHUNK), :], w_ref[:, lo:hi],
            preferred_element_type=jnp.float32,
        )

    barrier_sem = pltpu.get_barrier_semaphore()
    for nbr in (left, right):
        pl.semaphore_signal(
            barrier_sem, inc=1,
            device_id=(nbr,), device_id_type=pl.DeviceIdType.MESH,
        )
    pl.semaphore_wait(barrier_sem, 2)

    comm_cw[0] = pchunk(my, 0, HALF)
    comm_ccw[0] = pchunk(my, HALF, N_OUT)

    pending_stores = []
    for t in range(N_STEPS):
        s_slot = t % 2
        r_slot = (t + 1) % 2
        if t >= 1:
            pl.semaphore_wait(credit_cw, 1)
            pl.semaphore_wait(credit_ccw, 1)
        rdma_cw = pltpu.make_async_remote_copy(
            src_ref=comm_cw.at[s_slot], dst_ref=comm_cw.at[r_slot],
            send_sem=send_cw.at[s_slot], recv_sem=recv_cw.at[r_slot],
            device_id=(right,), device_id_type=pl.DeviceIdType.MESH,
        )
        rdma_ccw = pltpu.make_async_remote_copy(
            src_ref=comm_ccw.at[s_slot], dst_ref=comm_ccw.at[r_slot],
            send_sem=send_ccw.at[s_slot], recv_sem=recv_ccw.at[r_slot],
            device_id=(left,), device_id_type=pl.DeviceIdType.MESH,
        )
        rdma_cw.start()
        rdma_ccw.start()
        if t < N_DEV - 1:
            c_cw = lax.rem(my - (t + 1) + N_DEV, N_DEV)
            c_ccw = lax.rem(my + t + 1, N_DEV)
            pa = pchunk(c_cw, 0, HALF)
            pb = pchunk(c_ccw, HALF, N_OUT)
        rdma_cw.wait()
        rdma_ccw.wait()
        if pending_stores:
            for st in pending_stores:
                st.wait()
            pending_stores = []
        if t < N_STEPS - 1:
            pl.semaphore_signal(
                credit_cw, inc=1,
                device_id=(left,), device_id_type=pl.DeviceIdType.MESH,
            )
            pl.semaphore_signal(
                credit_ccw, inc=1,
                device_id=(right,), device_id_type=pl.DeviceIdType.MESH,
            )
        if t < N_DEV - 2:
            comm_cw[r_slot] = comm_cw[r_slot] + pa
            comm_ccw[r_slot] = comm_ccw[r_slot] + pb
        elif t == N_DEV - 2:
            comm_cw[r_slot] = _silu(comm_cw[r_slot] + pa)
            comm_ccw[r_slot] = _silu(comm_ccw[r_slot] + pb)
        if t >= N_DEV - 2:
            c_a = lax.rem(my - (t - (N_DEV - 1)) + N_DEV, N_DEV)
            c_b = lax.rem(my + (t - (N_DEV - 1)) + N_DEV, N_DEV)
            st_a = pltpu.make_async_copy(
                comm_cw.at[r_slot],
                out_hbm.at[pl.ds(c_a * CHUNK, CHUNK), pl.ds(0, HALF)],
                out_sem_a)
            st_b = pltpu.make_async_copy(
                comm_ccw.at[r_slot],
                out_hbm.at[pl.ds(c_b * CHUNK, CHUNK), pl.ds(HALF, HALF)],
                out_sem_b)
            st_a.start()
            st_b.start()
            pending_stores = [st_a, st_b]

    for st in pending_stores:
        st.wait()


def kernel(x, w_mat):
    return pl.pallas_call(
        _body,
        out_shape=jax.ShapeDtypeStruct((M, N_OUT), jnp.float32),
        in_specs=[
            pl.BlockSpec(memory_space=pltpu.VMEM),
            pl.BlockSpec(memory_space=pltpu.VMEM),
        ],
        out_specs=pl.BlockSpec(memory_space=pltpu.MemorySpace.HBM),
        scratch_shapes=[
            pltpu.VMEM((2, CHUNK, HALF), jnp.float32),
            pltpu.VMEM((2, CHUNK, HALF), jnp.float32),
            pltpu.SemaphoreType.DMA((2,)),
            pltpu.SemaphoreType.DMA((2,)),
            pltpu.SemaphoreType.DMA((2,)),
            pltpu.SemaphoreType.DMA((2,)),
            pltpu.SemaphoreType.DMA,
            pltpu.SemaphoreType.DMA,
            pltpu.SemaphoreType.REGULAR,
            pltpu.SemaphoreType.REGULAR,
        ],
        compiler_params=pltpu.CompilerParams(collective_id=0),
    )(x, w_mat)
